# baseline (speedup 1.0000x reference)
import functools

import jax
import jax.numpy as jnp
from jax import lax
from jax.experimental import pallas as pl
from jax.experimental.pallas import tpu as pltpu
from jax.experimental.pallas import tpu_sc as plsc

D = 64
B = 4096
L = 200
HID = 300
NCLS = 100
V = 1_000_000

NC = 2
NS = 16
NW = NC * NS

CW = 256
NCH_T = V // CW
TAIL0 = NCH_T * CW
TAILW = V - TAIL0
CPT = (NCH_T + NW - 1) // NW


def _tr_body(tt_hbm, tailw_hbm, out_hbm, ibuf, obuf, scr, isems, osems):
    wid = lax.axis_index("s") * NC + lax.axis_index("c")
    iot = lax.iota(jnp.int32, 16)
    sidx = [iot * 17 + r for r in range(16)]

    def chunk_of(k):
        return lax.min(wid + k * NW, NCH_T - 1)

    def in_cps(k, slot):
        c = chunk_of(k)
        cps = []
        for tr in range(D // 8):
            for tc in range(CW // 128):
                cps.append(pltpu.make_async_copy(
                    tt_hbm.at[pl.ds(tr * 8, 8), pl.ds(c * CW + tc * 128, 128)],
                    ibuf.at[slot, tr, tc], isems.at[slot]))
        return cps

    def out_cp(k, slot):
        c = chunk_of(k)
        return pltpu.make_async_copy(
            obuf.at[slot], out_hbm.at[pl.ds(c * (CW // 2), CW // 2)],
            osems.at[slot])

    def block_xpose(slot):
        def tc_body(tc, c2):
            def bd_body(bd, c3):
                d0 = bd * 16
                for bjm in range(8):
                    for r in range(16):
                        v = ibuf[slot, bd * 2 + r // 8, tc, r % 8,
                                 pl.ds(bjm * 16, 16)]
                        plsc.store_scatter(scr, [sidx[r]], v)
                    for jl in range(16):
                        tv = scr[pl.ds(jl * 17, 16)]
                        ql = lax.shift_right_logical(
                            tc * 128 + bjm * 16 + jl, 1)
                        obuf[slot, ql, pl.ds((jl & 1) * 64 + d0, 16)] = tv
                return c3
            lax.fori_loop(0, 4, bd_body, 0)
            return c2
        lax.fori_loop(0, CW // 128, tc_body, 0)

    for cp in in_cps(0, 0):
        cp.start()
    for cp in in_cps(1, 1):
        cp.start()

    def step(k, carry):
        slot = lax.rem(k, 2)
        for cp in in_cps(k, slot):
            cp.wait()

        @pl.when(k >= 2)
        def _():
            out_cp(k - 2, slot).wait()

        block_xpose(slot)

        @pl.when(k + 2 < CPT)
        def _():
            for cp in in_cps(k + 2, slot):
                cp.start()

        out_cp(k, slot).start()
        return carry

    lax.fori_loop(0, CPT, step, 0)
    out_cp(CPT - 2, lax.rem(CPT, 2)).wait()
    out_cp(CPT - 1, lax.rem(CPT + 1, 2)).wait()

    @pl.when(wid == 0)
    def _():
        pltpu.sync_copy(tailw_hbm, out_hbm.at[pl.ds(TAIL0 // 2, TAILW // 2)])


_transpose = functools.partial(
    pl.kernel,
    out_type=jax.ShapeDtypeStruct((V // 2, 2 * D), jnp.float32),
    mesh=plsc.VectorSubcoreMesh(core_axis_name="c", subcore_axis_name="s"),
    compiler_params=pltpu.CompilerParams(needs_layout_passes=False),
    scratch_types=[
        pltpu.VMEM((2, D // 8, CW // 128, 8, 128), jnp.float32),
        pltpu.VMEM((2, CW // 2, 2 * D), jnp.float32),
        pltpu.VMEM((272,), jnp.float32),
        pltpu.SemaphoreType.DMA((2,)),
        pltpu.SemaphoreType.DMA((2,)),
    ],
)(_tr_body)


ROWS_PER = B // NW
TOK_PER = ROWS_PER * L
C0, C1 = 104, 96
NBUF = 7


def _pool_body(x_hbm, table_hbm, out_hbm, idx_v, rows_v, acc_v, sems):
    wid = lax.axis_index("s") * NC + lax.axis_index("c")
    pltpu.sync_copy(x_hbm.at[wid], idx_v)

    def gather(r, h, slot):
        base = r * L + h * C0
        ln = C1 if h else C0
        return pltpu.make_async_copy(
            table_hbm.at[idx_v.at[pl.ds(base, ln)]],
            rows_v.at[slot, pl.ds(0, ln)], sems.at[slot])

    for j in range(NBUF - 1):
        gather(j // 2, j % 2, j % NBUF).start()

    def row_body(r, carry):
        accs = (jnp.zeros((16,), jnp.float32),) * 4
        for h in range(2):
            j = r * 2 + h
            slot = lax.rem(j, NBUF)
            gather(r, h, slot).wait()

            @pl.when(j + NBUF - 1 < 2 * ROWS_PER)
            def _():
                jn = j + NBUF - 1
                gather(jn // 2, h, lax.rem(jn, NBUF)).start()

            ln = C1 if h else C0

            def tacc(t, a):
                t0 = t * 8
                out = list(a)
                for u in range(8):
                    for k in range(4):
                        out[k] = out[k] + rows_v[slot, t0 + u,
                                                 pl.ds(k * 16, 16)]
                return tuple(out)

            accs = lax.fori_loop(0, ln // 8, tacc, accs)
        for k in range(4):
            acc_v[r, pl.ds(k * 16, 16)] = accs[k]
        return carry

    lax.fori_loop(0, ROWS_PER, row_body, 0)
    pltpu.sync_copy(acc_v, out_hbm.at[wid])


_pool = functools.partial(
    pl.kernel,
    out_type=jax.ShapeDtypeStruct((NW, ROWS_PER, D), jnp.float32),
    mesh=plsc.VectorSubcoreMesh(core_axis_name="c", subcore_axis_name="s"),
    compiler_params=pltpu.CompilerParams(use_tc_tiling_on_sc=False),
    scratch_types=[
        pltpu.VMEM((TOK_PER,), jnp.int32),
        pltpu.VMEM((NBUF, C0, D), jnp.float32),
        pltpu.VMEM((ROWS_PER, D), jnp.float32),
        pltpu.SemaphoreType.DMA((NBUF,)),
    ],
)(_pool_body)


def _mlp_body(p_ref, w1_ref, b1_ref, w2_ref, b2_ref, o_ref):
    h = jnp.dot(p_ref[...] * (1.0 / L), w1_ref[...],
                preferred_element_type=jnp.float32) + b1_ref[...]
    h = jnp.maximum(h, 0.0)
    o_ref[...] = jnp.dot(h, w2_ref[...],
                         preferred_element_type=jnp.float32) + b2_ref[...]


def _mlp(pooled, W1, b1, W2, b2):
    return pl.pallas_call(
        _mlp_body,
        out_shape=jax.ShapeDtypeStruct((B, NCLS), jnp.float32),
    )(pooled, W1, b1.reshape(1, HID), W2, b2.reshape(1, NCLS))


def kernel(x, table, W1, b1, W2, b2):
    x2 = x.astype(jnp.int32).reshape(NW, TOK_PER)
    tailw = table[TAIL0:, :].reshape(TAILW // 2, 2 * D)
    twide = _transpose(table.T, tailw)
    t64 = twide.reshape(V, D)
    pooled = _pool(x2, t64)
    return _mlp(pooled.reshape(B, D), W1, b1, W2, b2)


# speedup vs baseline: 1.4366x; 1.4366x over previous
import functools

import jax
import jax.numpy as jnp
from jax import lax
from jax.experimental import pallas as pl
from jax.experimental.pallas import tpu as pltpu
from jax.experimental.pallas import tpu_sc as plsc

D = 64
B = 4096
L = 200
HID = 300
NCLS = 100
V = 1_000_000

NC = 2
NS = 16
NW = NC * NS

CW = 256
NCH_T = V // CW
TAIL0 = NCH_T * CW
TAILW = V - TAIL0
CPT = (NCH_T + NW - 1) // NW


def _tr_body(tt_hbm, tailw_hbm, out_hbm, ibuf, obuf, scr, isems, osems):
    wid = lax.axis_index("s") * NC + lax.axis_index("c")
    iot = lax.iota(jnp.int32, 16)
    sidx = [iot * 17 + r for r in range(16)]

    def chunk_of(k):
        return lax.min(wid + k * NW, NCH_T - 1)

    def in_cps(k, slot):
        c = chunk_of(k)
        cps = []
        for tr in range(D // 8):
            for tc in range(CW // 128):
                cps.append(pltpu.make_async_copy(
                    tt_hbm.at[pl.ds(tr * 8, 8), pl.ds(c * CW + tc * 128, 128)],
                    ibuf.at[slot, tr, tc], isems.at[slot]))
        return cps

    def out_cp(k, slot):
        c = chunk_of(k)
        return pltpu.make_async_copy(
            obuf.at[slot], out_hbm.at[pl.ds(c * (CW // 2), CW // 2)],
            osems.at[slot])

    def block_xpose(slot):
        def tc_body(tc, c2):
            def bd_body(bd, c3):
                d0 = bd * 16
                for bjm in range(8):
                    for r in range(16):
                        v = ibuf[slot, bd * 2 + r // 8, tc, r % 8,
                                 pl.ds(bjm * 16, 16)]
                        plsc.store_scatter(scr, [sidx[r]], v)
                    for jl in range(16):
                        tv = scr[pl.ds(jl * 17, 16)]
                        ql = lax.shift_right_logical(
                            tc * 128 + bjm * 16 + jl, 1)
                        obuf[slot, ql, pl.ds((jl & 1) * 64 + d0, 16)] = tv
                return c3
            lax.fori_loop(0, 4, bd_body, 0)
            return c2
        lax.fori_loop(0, CW // 128, tc_body, 0)

    for cp in in_cps(0, 0):
        cp.start()
    for cp in in_cps(1, 1):
        cp.start()

    def step(k, carry):
        slot = lax.rem(k, 2)
        for cp in in_cps(k, slot):
            cp.wait()

        @pl.when(k >= 2)
        def _():
            out_cp(k - 2, slot).wait()

        block_xpose(slot)

        @pl.when(k + 2 < CPT)
        def _():
            for cp in in_cps(k + 2, slot):
                cp.start()

        out_cp(k, slot).start()
        return carry

    lax.fori_loop(0, CPT, step, 0)
    out_cp(CPT - 2, lax.rem(CPT, 2)).wait()
    out_cp(CPT - 1, lax.rem(CPT + 1, 2)).wait()

    @pl.when(wid == 0)
    def _():
        pltpu.sync_copy(tailw_hbm, out_hbm.at[pl.ds(TAIL0 // 2, TAILW // 2)])


_transpose = functools.partial(
    pl.kernel,
    out_type=jax.ShapeDtypeStruct((V // 2, 2 * D), jnp.float32),
    mesh=plsc.VectorSubcoreMesh(core_axis_name="c", subcore_axis_name="s"),
    compiler_params=pltpu.CompilerParams(needs_layout_passes=False),
    scratch_types=[
        pltpu.VMEM((2, D // 8, CW // 128, 8, 128), jnp.float32),
        pltpu.VMEM((2, CW // 2, 2 * D), jnp.float32),
        pltpu.VMEM((272,), jnp.float32),
        pltpu.SemaphoreType.DMA((2,)),
        pltpu.SemaphoreType.DMA((2,)),
    ],
)(_tr_body)


ROWS_PER = B // NW
TOK_PER = ROWS_PER * L
C0, C1 = 104, 96
NBUF = 7


def _pool_body(x_hbm, table_hbm, out_hbm, idx_v, rows_v, acc_v, sems):
    wid = lax.axis_index("s") * NC + lax.axis_index("c")
    pltpu.sync_copy(x_hbm.at[wid], idx_v)

    def gather(r, h, slot):
        base = r * L + h * C0
        ln = C1 if h else C0
        return pltpu.make_async_copy(
            table_hbm.at[idx_v.at[pl.ds(base, ln)]],
            rows_v.at[slot, pl.ds(0, ln)], sems.at[slot])

    for j in range(NBUF - 1):
        gather(j // 2, j % 2, j % NBUF).start()

    def row_body(r, carry):
        accs = (jnp.zeros((16,), jnp.float32),) * 4
        for h in range(2):
            j = r * 2 + h
            slot = lax.rem(j, NBUF)
            gather(r, h, slot).wait()

            @pl.when(j + NBUF - 1 < 2 * ROWS_PER)
            def _():
                jn = j + NBUF - 1
                gather(jn // 2, h, lax.rem(jn, NBUF)).start()

            ln = C1 if h else C0

            def tacc(t, a):
                t0 = t * 8
                out = list(a)
                for u in range(8):
                    for k in range(4):
                        out[k] = out[k] + rows_v[slot, t0 + u,
                                                 pl.ds(k * 16, 16)]
                return tuple(out)

            accs = lax.fori_loop(0, ln // 8, tacc, accs)
        for k in range(4):
            acc_v[r, pl.ds(k * 16, 16)] = accs[k]
        return carry

    lax.fori_loop(0, ROWS_PER, row_body, 0)
    pltpu.sync_copy(acc_v, out_hbm.at[wid])


_pool = functools.partial(
    pl.kernel,
    out_type=jax.ShapeDtypeStruct((NW, ROWS_PER, D), jnp.float32),
    mesh=plsc.VectorSubcoreMesh(core_axis_name="c", subcore_axis_name="s"),
    compiler_params=pltpu.CompilerParams(use_tc_tiling_on_sc=False),
    scratch_types=[
        pltpu.VMEM((TOK_PER,), jnp.int32),
        pltpu.VMEM((NBUF, C0, D), jnp.float32),
        pltpu.VMEM((ROWS_PER, D), jnp.float32),
        pltpu.SemaphoreType.DMA((NBUF,)),
    ],
)(_pool_body)


def _mlp_body(p_ref, w1_ref, b1_ref, w2_ref, b2_ref, o_ref):
    h = jnp.dot(p_ref[...] * (1.0 / L), w1_ref[...],
                preferred_element_type=jnp.float32) + b1_ref[...]
    h = jnp.maximum(h, 0.0)
    o_ref[...] = jnp.dot(h, w2_ref[...],
                         preferred_element_type=jnp.float32) + b2_ref[...]


def _mlp(pooled, W1, b1, W2, b2):
    return pl.pallas_call(
        _mlp_body,
        out_shape=jax.ShapeDtypeStruct((B, NCLS), jnp.float32),
    )(pooled, W1, b1.reshape(1, HID), W2, b2.reshape(1, NCLS))


def kernel(x, table, W1, b1, W2, b2):
    x2 = x.astype(jnp.int32).reshape(NW, TOK_PER)
    pooled = _pool(x2, table)
    return _mlp(pooled.reshape(B, D), W1, b1, W2, b2)


# speedup vs baseline: 2.3025x; 1.6028x over previous
import functools

import jax
import jax.numpy as jnp
from jax import lax
from jax.experimental import pallas as pl
from jax.experimental.pallas import tpu as pltpu
from jax.experimental.pallas import tpu_sc as plsc

D = 64
B = 4096
L = 200
HID = 300
NCLS = 100
V = 1_000_000

NC = 2
NS = 16
NW = NC * NS

CW = 256
NCH_T = V // CW
TAIL0 = NCH_T * CW
TAILW = V - TAIL0
CPT = (NCH_T + NW - 1) // NW


def _tr_body(tt_hbm, tailw_hbm, out_hbm, ibuf, obuf, scr, isems, osems):
    wid = lax.axis_index("s") * NC + lax.axis_index("c")
    iot = lax.iota(jnp.int32, 16)
    sidx = [iot * 17 + r for r in range(16)]

    def chunk_of(k):
        return lax.min(wid + k * NW, NCH_T - 1)

    def in_cps(k, slot):
        c = chunk_of(k)
        cps = []
        for tr in range(D // 8):
            for tc in range(CW // 128):
                cps.append(pltpu.make_async_copy(
                    tt_hbm.at[pl.ds(tr * 8, 8), pl.ds(c * CW + tc * 128, 128)],
                    ibuf.at[slot, tr, tc], isems.at[slot]))
        return cps

    def out_cp(k, slot):
        c = chunk_of(k)
        return pltpu.make_async_copy(
            obuf.at[slot], out_hbm.at[pl.ds(c * (CW // 2), CW // 2)],
            osems.at[slot])

    def block_xpose(slot):
        def tc_body(tc, c2):
            def bd_body(bd, c3):
                d0 = bd * 16
                for bjm in range(8):
                    for r in range(16):
                        v = ibuf[slot, bd * 2 + r // 8, tc, r % 8,
                                 pl.ds(bjm * 16, 16)]
                        plsc.store_scatter(scr, [sidx[r]], v)
                    for jl in range(16):
                        tv = scr[pl.ds(jl * 17, 16)]
                        ql = lax.shift_right_logical(
                            tc * 128 + bjm * 16 + jl, 1)
                        obuf[slot, ql, pl.ds((jl & 1) * 64 + d0, 16)] = tv
                return c3
            lax.fori_loop(0, 4, bd_body, 0)
            return c2
        lax.fori_loop(0, CW // 128, tc_body, 0)

    for cp in in_cps(0, 0):
        cp.start()
    for cp in in_cps(1, 1):
        cp.start()

    def step(k, carry):
        slot = lax.rem(k, 2)
        for cp in in_cps(k, slot):
            cp.wait()

        @pl.when(k >= 2)
        def _():
            out_cp(k - 2, slot).wait()

        block_xpose(slot)

        @pl.when(k + 2 < CPT)
        def _():
            for cp in in_cps(k + 2, slot):
                cp.start()

        out_cp(k, slot).start()
        return carry

    lax.fori_loop(0, CPT, step, 0)
    out_cp(CPT - 2, lax.rem(CPT, 2)).wait()
    out_cp(CPT - 1, lax.rem(CPT + 1, 2)).wait()

    @pl.when(wid == 0)
    def _():
        pltpu.sync_copy(tailw_hbm, out_hbm.at[pl.ds(TAIL0 // 2, TAILW // 2)])


_transpose = functools.partial(
    pl.kernel,
    out_type=jax.ShapeDtypeStruct((V // 2, 2 * D), jnp.float32),
    mesh=plsc.VectorSubcoreMesh(core_axis_name="c", subcore_axis_name="s"),
    compiler_params=pltpu.CompilerParams(needs_layout_passes=False),
    scratch_types=[
        pltpu.VMEM((2, D // 8, CW // 128, 8, 128), jnp.float32),
        pltpu.VMEM((2, CW // 2, 2 * D), jnp.float32),
        pltpu.VMEM((272,), jnp.float32),
        pltpu.SemaphoreType.DMA((2,)),
        pltpu.SemaphoreType.DMA((2,)),
    ],
)(_tr_body)


ROWS_PER = B // NW
TOK_PER = ROWS_PER * L
C0, C1 = 104, 96
NBUF = 7


def _pool_body(x_hbm, table_hbm, out_hbm, idx_v, rows_v, acc_v, sems):
    wid = lax.axis_index("s") * NC + lax.axis_index("c")
    pltpu.sync_copy(x_hbm.at[wid], idx_v)

    def gather(r, h, slot):
        base = r * L + h * C0
        ln = C1 if h else C0
        return pltpu.make_async_copy(
            table_hbm.at[idx_v.at[pl.ds(base, ln)]],
            rows_v.at[slot, pl.ds(0, ln)], sems.at[slot])

    for j in range(NBUF - 1):
        gather(j // 2, j % 2, j % NBUF).start()

    def row_body(r, carry):
        accs = (jnp.zeros((16,), jnp.float32),) * 4
        for h in range(2):
            j = r * 2 + h
            slot = lax.rem(j, NBUF)
            gather(r, h, slot).wait()

            @pl.when(j + NBUF - 1 < 2 * ROWS_PER)
            def _():
                jn = j + NBUF - 1
                gather(jn // 2, h, lax.rem(jn, NBUF)).start()

            ln = C1 if h else C0

            def tacc(t, a):
                t0 = t * 8
                out = list(a)
                for u in range(8):
                    for k in range(4):
                        out[k] = out[k] + rows_v[slot, t0 + u,
                                                 pl.ds(k * 16, 16)]
                return tuple(out)

            accs = lax.fori_loop(0, ln // 8, tacc, accs)
        for k in range(4):
            acc_v[r, pl.ds(k * 16, 16)] = accs[k]
        return carry

    lax.fori_loop(0, ROWS_PER, row_body, 0)
    pltpu.sync_copy(acc_v, out_hbm.at[wid])


_pool = functools.partial(
    pl.kernel,
    out_type=jax.ShapeDtypeStruct((NW, ROWS_PER, D), jnp.float32),
    mesh=plsc.VectorSubcoreMesh(core_axis_name="c", subcore_axis_name="s"),
    compiler_params=pltpu.CompilerParams(use_tc_tiling_on_sc=False),
    scratch_types=[
        pltpu.VMEM((TOK_PER,), jnp.int32),
        pltpu.VMEM((NBUF, C0, D), jnp.float32),
        pltpu.VMEM((ROWS_PER, D), jnp.float32),
        pltpu.SemaphoreType.DMA((NBUF,)),
    ],
)(_pool_body)


def _mlp_body(p_ref, w1_ref, b1_ref, w2_ref, b2_ref, o_ref):
    h = jnp.dot(p_ref[...] * (1.0 / L), w1_ref[...],
                preferred_element_type=jnp.float32) + b1_ref[...]
    h = jnp.maximum(h, 0.0)
    o_ref[...] = jnp.dot(h, w2_ref[...],
                         preferred_element_type=jnp.float32) + b2_ref[...]


def _mlp(pooled, W1, b1, W2, b2):
    return pl.pallas_call(
        _mlp_body,
        out_shape=jax.ShapeDtypeStruct((B, NCLS), jnp.float32),
    )(pooled, W1, b1.reshape(1, HID), W2, b2.reshape(1, NCLS))


TBLK = 4096
NTBLK = (V + TBLK - 1) // TBLK
HB = TBLK // 2
VP = NTBLK * TBLK


def _tr_tc_body(tt_ref, o_ref):
    xt = tt_ref[...].T
    o_ref[:, 0:D] = xt[0:HB, :]
    o_ref[:, D:2 * D] = xt[HB:TBLK, :]


def _tc_transpose(tt):
    return pl.pallas_call(
        _tr_tc_body,
        grid=(NTBLK,),
        in_specs=[pl.BlockSpec((D, TBLK), lambda i: (0, i))],
        out_specs=pl.BlockSpec((HB, 2 * D), lambda i: (i, 0)),
        out_shape=jax.ShapeDtypeStruct((NTBLK * HB, 2 * D), jnp.float32),
    )(tt)


def kernel(x, table, W1, b1, W2, b2):
    xi = x.astype(jnp.int32)
    r = xi & (TBLK - 1)
    gidx = (xi - r) + jnp.where(r < HB, 2 * r, 2 * (r - HB) + 1)
    x2 = gidx.reshape(NW, TOK_PER)
    twide = _tc_transpose(table.T)
    t64 = twide.reshape(VP, D)
    pooled = _pool(x2, t64)
    return _mlp(pooled.reshape(B, D), W1, b1, W2, b2)


# speedup vs baseline: 2.7215x; 1.1819x over previous
import functools

import jax
import jax.numpy as jnp
from jax import lax
from jax.experimental import pallas as pl
from jax.experimental.pallas import tpu as pltpu
from jax.experimental.pallas import tpu_sc as plsc

D = 64
B = 4096
L = 200
HID = 300
NCLS = 100
V = 1_000_000

NC = 2
NS = 16
NW = NC * NS

CW = 256
NCH_T = V // CW
TAIL0 = NCH_T * CW
TAILW = V - TAIL0
CPT = (NCH_T + NW - 1) // NW


def _tr_body(tt_hbm, tailw_hbm, out_hbm, ibuf, obuf, scr, isems, osems):
    wid = lax.axis_index("s") * NC + lax.axis_index("c")
    iot = lax.iota(jnp.int32, 16)
    sidx = [iot * 17 + r for r in range(16)]

    def chunk_of(k):
        return lax.min(wid + k * NW, NCH_T - 1)

    def in_cps(k, slot):
        c = chunk_of(k)
        cps = []
        for tr in range(D // 8):
            for tc in range(CW // 128):
                cps.append(pltpu.make_async_copy(
                    tt_hbm.at[pl.ds(tr * 8, 8), pl.ds(c * CW + tc * 128, 128)],
                    ibuf.at[slot, tr, tc], isems.at[slot]))
        return cps

    def out_cp(k, slot):
        c = chunk_of(k)
        return pltpu.make_async_copy(
            obuf.at[slot], out_hbm.at[pl.ds(c * (CW // 2), CW // 2)],
            osems.at[slot])

    def block_xpose(slot):
        def tc_body(tc, c2):
            def bd_body(bd, c3):
                d0 = bd * 16
                for bjm in range(8):
                    for r in range(16):
                        v = ibuf[slot, bd * 2 + r // 8, tc, r % 8,
                                 pl.ds(bjm * 16, 16)]
                        plsc.store_scatter(scr, [sidx[r]], v)
                    for jl in range(16):
                        tv = scr[pl.ds(jl * 17, 16)]
                        ql = lax.shift_right_logical(
                            tc * 128 + bjm * 16 + jl, 1)
                        obuf[slot, ql, pl.ds((jl & 1) * 64 + d0, 16)] = tv
                return c3
            lax.fori_loop(0, 4, bd_body, 0)
            return c2
        lax.fori_loop(0, CW // 128, tc_body, 0)

    for cp in in_cps(0, 0):
        cp.start()
    for cp in in_cps(1, 1):
        cp.start()

    def step(k, carry):
        slot = lax.rem(k, 2)
        for cp in in_cps(k, slot):
            cp.wait()

        @pl.when(k >= 2)
        def _():
            out_cp(k - 2, slot).wait()

        block_xpose(slot)

        @pl.when(k + 2 < CPT)
        def _():
            for cp in in_cps(k + 2, slot):
                cp.start()

        out_cp(k, slot).start()
        return carry

    lax.fori_loop(0, CPT, step, 0)
    out_cp(CPT - 2, lax.rem(CPT, 2)).wait()
    out_cp(CPT - 1, lax.rem(CPT + 1, 2)).wait()

    @pl.when(wid == 0)
    def _():
        pltpu.sync_copy(tailw_hbm, out_hbm.at[pl.ds(TAIL0 // 2, TAILW // 2)])


_transpose = functools.partial(
    pl.kernel,
    out_type=jax.ShapeDtypeStruct((V // 2, 2 * D), jnp.float32),
    mesh=plsc.VectorSubcoreMesh(core_axis_name="c", subcore_axis_name="s"),
    compiler_params=pltpu.CompilerParams(needs_layout_passes=False),
    scratch_types=[
        pltpu.VMEM((2, D // 8, CW // 128, 8, 128), jnp.float32),
        pltpu.VMEM((2, CW // 2, 2 * D), jnp.float32),
        pltpu.VMEM((272,), jnp.float32),
        pltpu.SemaphoreType.DMA((2,)),
        pltpu.SemaphoreType.DMA((2,)),
    ],
)(_tr_body)


ROWS_PER = B // NW
TOK_PER = ROWS_PER * L
C0, C1 = 104, 96
NBUF = 7


def _pool_body(x_hbm, table_hbm, out_hbm, idx_v, rows_v, acc_v, sems):
    wid = lax.axis_index("s") * NC + lax.axis_index("c")
    pltpu.sync_copy(x_hbm.at[wid], idx_v)

    def gather(r, h, slot):
        base = r * L + h * C0
        ln = C1 if h else C0
        return pltpu.make_async_copy(
            table_hbm.at[idx_v.at[pl.ds(base, ln)]],
            rows_v.at[slot, pl.ds(0, ln)], sems.at[slot])

    for j in range(NBUF - 1):
        gather(j // 2, j % 2, j % NBUF).start()

    def row_body(r, carry):
        accs = (jnp.zeros((16,), jnp.float32),) * 4
        for h in range(2):
            j = r * 2 + h
            slot = lax.rem(j, NBUF)
            gather(r, h, slot).wait()

            @pl.when(j + NBUF - 1 < 2 * ROWS_PER)
            def _():
                jn = j + NBUF - 1
                gather(jn // 2, h, lax.rem(jn, NBUF)).start()

            ln = C1 if h else C0

            def tacc(t, a):
                t0 = t * 8
                out = list(a)
                for u in range(8):
                    for k in range(4):
                        out[k] = out[k] + rows_v[slot, t0 + u,
                                                 pl.ds(k * 16, 16)]
                return tuple(out)

            accs = lax.fori_loop(0, ln // 8, tacc, accs)
        for k in range(4):
            acc_v[r, pl.ds(k * 16, 16)] = accs[k]
        return carry

    lax.fori_loop(0, ROWS_PER, row_body, 0)
    pltpu.sync_copy(acc_v, out_hbm.at[wid])


_pool = functools.partial(
    pl.kernel,
    out_type=jax.ShapeDtypeStruct((NW, ROWS_PER, D), jnp.float32),
    mesh=plsc.VectorSubcoreMesh(core_axis_name="c", subcore_axis_name="s"),
    compiler_params=pltpu.CompilerParams(use_tc_tiling_on_sc=False),
    scratch_types=[
        pltpu.VMEM((TOK_PER,), jnp.int32),
        pltpu.VMEM((NBUF, C0, D), jnp.float32),
        pltpu.VMEM((ROWS_PER, D), jnp.float32),
        pltpu.SemaphoreType.DMA((NBUF,)),
    ],
)(_pool_body)


def _mlp_body(p_ref, w1_ref, b1_ref, w2_ref, b2_ref, o_ref):
    h = jnp.dot(p_ref[...] * (1.0 / L), w1_ref[...],
                preferred_element_type=jnp.float32) + b1_ref[...]
    h = jnp.maximum(h, 0.0)
    o_ref[...] = jnp.dot(h, w2_ref[...],
                         preferred_element_type=jnp.float32) + b2_ref[...]


def _mlp(pooled, W1, b1, W2, b2):
    return pl.pallas_call(
        _mlp_body,
        out_shape=jax.ShapeDtypeStruct((B, NCLS), jnp.float32),
    )(pooled, W1, b1.reshape(1, HID), W2, b2.reshape(1, NCLS))


TBLK = 8192
NTBLK = (V + TBLK - 1) // TBLK
HB = TBLK // 2
VP = NTBLK * TBLK


def _tr_tc_body(tt_ref, o_ref):
    xt = tt_ref[...].T
    o_ref[:, 0:D] = xt[0:HB, :]
    o_ref[:, D:2 * D] = xt[HB:TBLK, :]


def _tc_transpose(tt):
    return pl.pallas_call(
        _tr_tc_body,
        grid=(NTBLK,),
        in_specs=[pl.BlockSpec((D, TBLK), lambda i: (0, i))],
        out_specs=pl.BlockSpec((HB, 2 * D), lambda i: (i, 0)),
        out_shape=jax.ShapeDtypeStruct((NTBLK * HB, 2 * D), jnp.float32),
        compiler_params=pltpu.CompilerParams(
            dimension_semantics=("parallel",)),
    )(tt)


def kernel(x, table, W1, b1, W2, b2):
    xi = x.astype(jnp.int32)
    r = xi & (TBLK - 1)
    gidx = (xi - r) + jnp.where(r < HB, 2 * r, 2 * (r - HB) + 1)
    x2 = gidx.reshape(NW, TOK_PER)
    twide = _tc_transpose(table.T)
    t64 = twide.reshape(VP, D)
    pooled = _pool(x2, t64)
    return _mlp(pooled.reshape(B, D), W1, b1, W2, b2)


# speedup vs baseline: 2.9960x; 1.1009x over previous
import functools

import jax
import jax.numpy as jnp
from jax import lax
from jax.experimental import pallas as pl
from jax.experimental.pallas import tpu as pltpu
from jax.experimental.pallas import tpu_sc as plsc

D = 64
B = 4096
L = 200
HID = 300
NCLS = 100
V = 1_000_000

NC = 2
NS = 16
NW = NC * NS

CW = 256
NCH_T = V // CW
TAIL0 = NCH_T * CW
TAILW = V - TAIL0
CPT = (NCH_T + NW - 1) // NW


def _tr_body(tt_hbm, tailw_hbm, out_hbm, ibuf, obuf, scr, isems, osems):
    wid = lax.axis_index("s") * NC + lax.axis_index("c")
    iot = lax.iota(jnp.int32, 16)
    sidx = [iot * 17 + r for r in range(16)]

    def chunk_of(k):
        return lax.min(wid + k * NW, NCH_T - 1)

    def in_cps(k, slot):
        c = chunk_of(k)
        cps = []
        for tr in range(D // 8):
            for tc in range(CW // 128):
                cps.append(pltpu.make_async_copy(
                    tt_hbm.at[pl.ds(tr * 8, 8), pl.ds(c * CW + tc * 128, 128)],
                    ibuf.at[slot, tr, tc], isems.at[slot]))
        return cps

    def out_cp(k, slot):
        c = chunk_of(k)
        return pltpu.make_async_copy(
            obuf.at[slot], out_hbm.at[pl.ds(c * (CW // 2), CW // 2)],
            osems.at[slot])

    def block_xpose(slot):
        def tc_body(tc, c2):
            def bd_body(bd, c3):
                d0 = bd * 16
                for bjm in range(8):
                    for r in range(16):
                        v = ibuf[slot, bd * 2 + r // 8, tc, r % 8,
                                 pl.ds(bjm * 16, 16)]
                        plsc.store_scatter(scr, [sidx[r]], v)
                    for jl in range(16):
                        tv = scr[pl.ds(jl * 17, 16)]
                        ql = lax.shift_right_logical(
                            tc * 128 + bjm * 16 + jl, 1)
                        obuf[slot, ql, pl.ds((jl & 1) * 64 + d0, 16)] = tv
                return c3
            lax.fori_loop(0, 4, bd_body, 0)
            return c2
        lax.fori_loop(0, CW // 128, tc_body, 0)

    for cp in in_cps(0, 0):
        cp.start()
    for cp in in_cps(1, 1):
        cp.start()

    def step(k, carry):
        slot = lax.rem(k, 2)
        for cp in in_cps(k, slot):
            cp.wait()

        @pl.when(k >= 2)
        def _():
            out_cp(k - 2, slot).wait()

        block_xpose(slot)

        @pl.when(k + 2 < CPT)
        def _():
            for cp in in_cps(k + 2, slot):
                cp.start()

        out_cp(k, slot).start()
        return carry

    lax.fori_loop(0, CPT, step, 0)
    out_cp(CPT - 2, lax.rem(CPT, 2)).wait()
    out_cp(CPT - 1, lax.rem(CPT + 1, 2)).wait()

    @pl.when(wid == 0)
    def _():
        pltpu.sync_copy(tailw_hbm, out_hbm.at[pl.ds(TAIL0 // 2, TAILW // 2)])


_transpose = functools.partial(
    pl.kernel,
    out_type=jax.ShapeDtypeStruct((V // 2, 2 * D), jnp.float32),
    mesh=plsc.VectorSubcoreMesh(core_axis_name="c", subcore_axis_name="s"),
    compiler_params=pltpu.CompilerParams(needs_layout_passes=False),
    scratch_types=[
        pltpu.VMEM((2, D // 8, CW // 128, 8, 128), jnp.float32),
        pltpu.VMEM((2, CW // 2, 2 * D), jnp.float32),
        pltpu.VMEM((272,), jnp.float32),
        pltpu.SemaphoreType.DMA((2,)),
        pltpu.SemaphoreType.DMA((2,)),
    ],
)(_tr_body)


ROWS_PER = B // NW
TOK_PER = ROWS_PER * L
C0, C1 = 104, 96
NBUF = 7


def _pool_body(x_hbm, table_hbm, out_hbm, idx_v, rows_v, acc_v, sems):
    wid = lax.axis_index("s") * NC + lax.axis_index("c")
    pltpu.sync_copy(x_hbm.at[wid], idx_v)

    def gather(r, h, slot):
        base = r * L + h * C0
        ln = C1 if h else C0
        return pltpu.make_async_copy(
            table_hbm.at[idx_v.at[pl.ds(base, ln)]],
            rows_v.at[slot, pl.ds(0, ln)], sems.at[slot])

    for j in range(NBUF - 1):
        gather(j // 2, j % 2, j % NBUF).start()

    def row_body(r, carry):
        accs = (jnp.zeros((16,), jnp.float32),) * 4
        for h in range(2):
            j = r * 2 + h
            slot = lax.rem(j, NBUF)
            gather(r, h, slot).wait()

            @pl.when(j + NBUF - 1 < 2 * ROWS_PER)
            def _():
                jn = j + NBUF - 1
                gather(jn // 2, h, lax.rem(jn, NBUF)).start()

            ln = C1 if h else C0

            def tacc(t, a):
                t0 = t * 8
                out = list(a)
                for u in range(8):
                    for k in range(4):
                        out[k] = out[k] + rows_v[slot, t0 + u,
                                                 pl.ds(k * 16, 16)]
                return tuple(out)

            accs = lax.fori_loop(0, ln // 8, tacc, accs)
        for k in range(4):
            acc_v[r, pl.ds(k * 16, 16)] = accs[k]
        return carry

    lax.fori_loop(0, ROWS_PER, row_body, 0)
    pltpu.sync_copy(acc_v, out_hbm.at[wid])


_pool = functools.partial(
    pl.kernel,
    out_type=jax.ShapeDtypeStruct((NW, ROWS_PER, D), jnp.float32),
    mesh=plsc.VectorSubcoreMesh(core_axis_name="c", subcore_axis_name="s"),
    compiler_params=pltpu.CompilerParams(use_tc_tiling_on_sc=False),
    scratch_types=[
        pltpu.VMEM((TOK_PER,), jnp.int32),
        pltpu.VMEM((NBUF, C0, D), jnp.float32),
        pltpu.VMEM((ROWS_PER, D), jnp.float32),
        pltpu.SemaphoreType.DMA((NBUF,)),
    ],
)(_pool_body)


def _mlp_body(p_ref, w1_ref, b1_ref, w2_ref, b2_ref, o_ref):
    h = jnp.dot(p_ref[...] * (1.0 / L), w1_ref[...],
                preferred_element_type=jnp.float32) + b1_ref[...]
    h = jnp.maximum(h, 0.0)
    o_ref[...] = jnp.dot(h, w2_ref[...],
                         preferred_element_type=jnp.float32) + b2_ref[...]


def _mlp(pooled, W1, b1, W2, b2):
    return pl.pallas_call(
        _mlp_body,
        out_shape=jax.ShapeDtypeStruct((B, NCLS), jnp.float32),
    )(pooled, W1, b1.reshape(1, HID), W2, b2.reshape(1, NCLS))


TBLK = 16384
NTBLK = (V + TBLK - 1) // TBLK
HB = TBLK // 2
VP = NTBLK * TBLK


def _tr_tc_body(tt_ref, o_ref):
    xt = tt_ref[...].T
    o_ref[:, 0:D] = xt[0:HB, :]
    o_ref[:, D:2 * D] = xt[HB:TBLK, :]


def _tc_transpose(tt):
    return pl.pallas_call(
        _tr_tc_body,
        grid=(NTBLK,),
        in_specs=[pl.BlockSpec((D, TBLK), lambda i: (0, i))],
        out_specs=pl.BlockSpec((HB, 2 * D), lambda i: (i, 0)),
        out_shape=jax.ShapeDtypeStruct((NTBLK * HB, 2 * D), jnp.float32),
        compiler_params=pltpu.CompilerParams(
            dimension_semantics=("parallel",)),
    )(tt)


def kernel(x, table, W1, b1, W2, b2):
    xi = x.astype(jnp.int32)
    r = xi & (TBLK - 1)
    gidx = (xi - r) + jnp.where(r < HB, 2 * r, 2 * (r - HB) + 1)
    x2 = gidx.reshape(NW, TOK_PER)
    twide = _tc_transpose(table.T)
    t64 = twide.reshape(VP, D)
    pooled = _pool(x2, t64)
    return _mlp(pooled.reshape(B, D), W1, b1, W2, b2)


# speedup vs baseline: 3.1309x; 1.0450x over previous
import functools

import jax
import jax.numpy as jnp
from jax import lax
from jax.experimental import pallas as pl
from jax.experimental.pallas import tpu as pltpu
from jax.experimental.pallas import tpu_sc as plsc

D = 64
B = 4096
L = 200
HID = 300
NCLS = 100
V = 1_000_000

NC = 2
NS = 16
NW = NC * NS

CW = 256
NCH_T = V // CW
TAIL0 = NCH_T * CW
TAILW = V - TAIL0
CPT = (NCH_T + NW - 1) // NW


def _tr_body(tt_hbm, tailw_hbm, out_hbm, ibuf, obuf, scr, isems, osems):
    wid = lax.axis_index("s") * NC + lax.axis_index("c")
    iot = lax.iota(jnp.int32, 16)
    sidx = [iot * 17 + r for r in range(16)]

    def chunk_of(k):
        return lax.min(wid + k * NW, NCH_T - 1)

    def in_cps(k, slot):
        c = chunk_of(k)
        cps = []
        for tr in range(D // 8):
            for tc in range(CW // 128):
                cps.append(pltpu.make_async_copy(
                    tt_hbm.at[pl.ds(tr * 8, 8), pl.ds(c * CW + tc * 128, 128)],
                    ibuf.at[slot, tr, tc], isems.at[slot]))
        return cps

    def out_cp(k, slot):
        c = chunk_of(k)
        return pltpu.make_async_copy(
            obuf.at[slot], out_hbm.at[pl.ds(c * (CW // 2), CW // 2)],
            osems.at[slot])

    def block_xpose(slot):
        def tc_body(tc, c2):
            def bd_body(bd, c3):
                d0 = bd * 16
                for bjm in range(8):
                    for r in range(16):
                        v = ibuf[slot, bd * 2 + r // 8, tc, r % 8,
                                 pl.ds(bjm * 16, 16)]
                        plsc.store_scatter(scr, [sidx[r]], v)
                    for jl in range(16):
                        tv = scr[pl.ds(jl * 17, 16)]
                        ql = lax.shift_right_logical(
                            tc * 128 + bjm * 16 + jl, 1)
                        obuf[slot, ql, pl.ds((jl & 1) * 64 + d0, 16)] = tv
                return c3
            lax.fori_loop(0, 4, bd_body, 0)
            return c2
        lax.fori_loop(0, CW // 128, tc_body, 0)

    for cp in in_cps(0, 0):
        cp.start()
    for cp in in_cps(1, 1):
        cp.start()

    def step(k, carry):
        slot = lax.rem(k, 2)
        for cp in in_cps(k, slot):
            cp.wait()

        @pl.when(k >= 2)
        def _():
            out_cp(k - 2, slot).wait()

        block_xpose(slot)

        @pl.when(k + 2 < CPT)
        def _():
            for cp in in_cps(k + 2, slot):
                cp.start()

        out_cp(k, slot).start()
        return carry

    lax.fori_loop(0, CPT, step, 0)
    out_cp(CPT - 2, lax.rem(CPT, 2)).wait()
    out_cp(CPT - 1, lax.rem(CPT + 1, 2)).wait()

    @pl.when(wid == 0)
    def _():
        pltpu.sync_copy(tailw_hbm, out_hbm.at[pl.ds(TAIL0 // 2, TAILW // 2)])


_transpose = functools.partial(
    pl.kernel,
    out_type=jax.ShapeDtypeStruct((V // 2, 2 * D), jnp.float32),
    mesh=plsc.VectorSubcoreMesh(core_axis_name="c", subcore_axis_name="s"),
    compiler_params=pltpu.CompilerParams(needs_layout_passes=False),
    scratch_types=[
        pltpu.VMEM((2, D // 8, CW // 128, 8, 128), jnp.float32),
        pltpu.VMEM((2, CW // 2, 2 * D), jnp.float32),
        pltpu.VMEM((272,), jnp.float32),
        pltpu.SemaphoreType.DMA((2,)),
        pltpu.SemaphoreType.DMA((2,)),
    ],
)(_tr_body)


ROWS_PER = B // NW
TOK_PER = ROWS_PER * L
C0, C1 = 104, 96
NBUF = 7


def _pool_body(x_hbm, table_hbm, out_hbm, idx_v, rows_v, acc_v, sems):
    wid = lax.axis_index("s") * NC + lax.axis_index("c")
    pltpu.sync_copy(x_hbm.at[wid], idx_v)

    def gather(r, h, slot):
        base = r * L + h * C0
        ln = C1 if h else C0
        return pltpu.make_async_copy(
            table_hbm.at[idx_v.at[pl.ds(base, ln)]],
            rows_v.at[slot, pl.ds(0, ln)], sems.at[slot])

    for j in range(NBUF - 1):
        gather(j // 2, j % 2, j % NBUF).start()

    def row_body(r, carry):
        accs = (jnp.zeros((16,), jnp.float32),) * 4
        for h in range(2):
            j = r * 2 + h
            slot = lax.rem(j, NBUF)
            gather(r, h, slot).wait()

            @pl.when(j + NBUF - 1 < 2 * ROWS_PER)
            def _():
                jn = j + NBUF - 1
                gather(jn // 2, h, lax.rem(jn, NBUF)).start()

            ln = C1 if h else C0

            def tacc(t, a):
                t0 = t * 8
                out = list(a)
                for u in range(8):
                    for k in range(4):
                        out[k] = out[k] + rows_v[slot, t0 + u,
                                                 pl.ds(k * 16, 16)]
                return tuple(out)

            accs = lax.fori_loop(0, ln // 8, tacc, accs)
        for k in range(4):
            acc_v[r, pl.ds(k * 16, 16)] = accs[k]
        return carry

    lax.fori_loop(0, ROWS_PER, row_body, 0)
    pltpu.sync_copy(acc_v, out_hbm.at[wid])


_pool = functools.partial(
    pl.kernel,
    out_type=jax.ShapeDtypeStruct((NW, ROWS_PER, D), jnp.float32),
    mesh=plsc.VectorSubcoreMesh(core_axis_name="c", subcore_axis_name="s"),
    compiler_params=pltpu.CompilerParams(use_tc_tiling_on_sc=False),
    scratch_types=[
        pltpu.VMEM((TOK_PER,), jnp.int32),
        pltpu.VMEM((NBUF, C0, D), jnp.float32),
        pltpu.VMEM((ROWS_PER, D), jnp.float32),
        pltpu.SemaphoreType.DMA((NBUF,)),
    ],
)(_pool_body)


def _mlp_body(p_ref, w1_ref, b1_ref, w2_ref, b2_ref, o_ref):
    h = jnp.dot(p_ref[...] * (1.0 / L), w1_ref[...],
                preferred_element_type=jnp.float32) + b1_ref[...]
    h = jnp.maximum(h, 0.0)
    o_ref[...] = jnp.dot(h, w2_ref[...],
                         preferred_element_type=jnp.float32) + b2_ref[...]


def _mlp(pooled, W1, b1, W2, b2):
    return pl.pallas_call(
        _mlp_body,
        out_shape=jax.ShapeDtypeStruct((B, NCLS), jnp.float32),
    )(pooled, W1, b1.reshape(1, HID), W2, b2.reshape(1, NCLS))


TBLK = 32768
NTBLK = (V + TBLK - 1) // TBLK
HB = TBLK // 2
VP = NTBLK * TBLK


def _tr_tc_body(tt_ref, o_ref):
    xt = tt_ref[...].T
    o_ref[:, 0:D] = xt[0:HB, :]
    o_ref[:, D:2 * D] = xt[HB:TBLK, :]


def _tc_transpose(tt):
    return pl.pallas_call(
        _tr_tc_body,
        grid=(NTBLK,),
        in_specs=[pl.BlockSpec((D, TBLK), lambda i: (0, i))],
        out_specs=pl.BlockSpec((HB, 2 * D), lambda i: (i, 0)),
        out_shape=jax.ShapeDtypeStruct((NTBLK * HB, 2 * D), jnp.float32),
        compiler_params=pltpu.CompilerParams(
            dimension_semantics=("parallel",)),
    )(tt)


def kernel(x, table, W1, b1, W2, b2):
    xi = x.astype(jnp.int32)
    r = xi & (TBLK - 1)
    gidx = (xi - r) + jnp.where(r < HB, 2 * r, 2 * (r - HB) + 1)
    x2 = gidx.reshape(NW, TOK_PER)
    twide = _tc_transpose(table.T)
    t64 = twide.reshape(VP, D)
    pooled = _pool(x2, t64)
    return _mlp(pooled.reshape(B, D), W1, b1, W2, b2)


# speedup vs baseline: 3.1355x; 1.0015x over previous
import functools

import jax
import jax.numpy as jnp
from jax import lax
from jax.experimental import pallas as pl
from jax.experimental.pallas import tpu as pltpu
from jax.experimental.pallas import tpu_sc as plsc

D = 64
B = 4096
L = 200
HID = 300
NCLS = 100
V = 1_000_000

NC = 2
NS = 16
NW = NC * NS

ROWS_PER = B // NW
TOK_PER = ROWS_PER * L
C0, C1 = 104, 96
NBUF = 7


def _pool_body(x_hbm, table_hbm, out_hbm, idx_v, rows_v, acc_v, sems):
    wid = lax.axis_index("s") * NC + lax.axis_index("c")
    pltpu.sync_copy(x_hbm.at[wid], idx_v)

    def gather(r, h, slot):
        base = r * L + h * C0
        ln = C1 if h else C0
        return pltpu.make_async_copy(
            table_hbm.at[idx_v.at[pl.ds(base, ln)]],
            rows_v.at[slot, pl.ds(0, ln)], sems.at[slot])

    for j in range(NBUF - 1):
        gather(j // 2, j % 2, j % NBUF).start()

    def row_body(r, carry):
        accs = (jnp.zeros((16,), jnp.float32),) * 4
        for h in range(2):
            j = r * 2 + h
            slot = lax.rem(j, NBUF)
            gather(r, h, slot).wait()

            @pl.when(j + NBUF - 1 < 2 * ROWS_PER)
            def _():
                jn = j + NBUF - 1
                gather(jn // 2, h, lax.rem(jn, NBUF)).start()

            ln = C1 if h else C0

            def tacc(t, a):
                t0 = t * 8
                out = list(a)
                for u in range(8):
                    for k in range(4):
                        out[k] = out[k] + rows_v[slot, t0 + u,
                                                 pl.ds(k * 16, 16)]
                return tuple(out)

            accs = lax.fori_loop(0, ln // 8, tacc, accs)
        for k in range(4):
            acc_v[r, pl.ds(k * 16, 16)] = accs[k]
        return carry

    lax.fori_loop(0, ROWS_PER, row_body, 0)
    pltpu.sync_copy(acc_v, out_hbm.at[wid])


_pool = functools.partial(
    pl.kernel,
    out_type=jax.ShapeDtypeStruct((NW, ROWS_PER, D), jnp.float32),
    mesh=plsc.VectorSubcoreMesh(core_axis_name="c", subcore_axis_name="s"),
    compiler_params=pltpu.CompilerParams(use_tc_tiling_on_sc=False),
    scratch_types=[
        pltpu.VMEM((TOK_PER,), jnp.int32),
        pltpu.VMEM((NBUF, C0, D), jnp.float32),
        pltpu.VMEM((ROWS_PER, D), jnp.float32),
        pltpu.SemaphoreType.DMA((NBUF,)),
    ],
)(_pool_body)


def _mlp_body(p_ref, w1_ref, b1_ref, w2_ref, b2_ref, o_ref):
    h = jnp.dot(p_ref[...] * (1.0 / L), w1_ref[...],
                preferred_element_type=jnp.float32) + b1_ref[...]
    h = jnp.maximum(h, 0.0)
    o_ref[...] = jnp.dot(h, w2_ref[...],
                         preferred_element_type=jnp.float32) + b2_ref[...]


def _mlp(pooled, W1, b1, W2, b2):
    return pl.pallas_call(
        _mlp_body,
        out_shape=jax.ShapeDtypeStruct((B, NCLS), jnp.float32),
    )(pooled, W1, b1.reshape(1, HID), W2, b2.reshape(1, NCLS))


TBLK = 32768
NTBLK = (V + TBLK - 1) // TBLK
HB = TBLK // 2
VP = NTBLK * TBLK


def _tr_tc_body(tt_ref, o_ref):
    xt = tt_ref[...].T
    o_ref[:, 0:D] = xt[0:HB, :]
    o_ref[:, D:2 * D] = xt[HB:TBLK, :]


def _tc_transpose(tt):
    return pl.pallas_call(
        _tr_tc_body,
        grid=(NTBLK,),
        in_specs=[pl.BlockSpec((D, TBLK), lambda i: (0, i))],
        out_specs=pl.BlockSpec((HB, 2 * D), lambda i: (i, 0)),
        out_shape=jax.ShapeDtypeStruct((NTBLK * HB, 2 * D), jnp.float32),
        compiler_params=pltpu.CompilerParams(
            dimension_semantics=("parallel",)),
    )(tt)


def kernel(x, table, W1, b1, W2, b2):
    xi = x.astype(jnp.int32)
    r = xi & (TBLK - 1)
    gidx = (xi - r) + jnp.where(r < HB, 2 * r, 2 * (r - HB) + 1)
    x2 = gidx.reshape(NW, TOK_PER)
    twide = _tc_transpose(table.T)
    t64 = twide.reshape(VP, D)
    pooled = _pool(x2, t64)
    return _mlp(pooled.reshape(B, D), W1, b1, W2, b2)
